# baseline (speedup 1.0000x reference)
import functools

import jax
import jax.numpy as jnp
from jax import lax
from jax.experimental import pallas as pl
from jax.experimental.pallas import tpu as pltpu
from jax.experimental.pallas import tpu_sc as plsc

NUM_CORES = 2
NUM_SUBCORES = 16
NW = NUM_CORES * NUM_SUBCORES
LANES = 16


@jax.jit
def _embed(idx4, table_t):
    h8_n, bt_n, hi_n, bi_n = idx4.shape
    hist = h8_n * hi_n
    d = 64
    vocab = table_t.shape[0] // d
    dt_n = d // 8
    bt_per_w = bt_n // NW
    assert h8_n % 2 == 1 and h8_n >= 3 and bt_n % NW == 0

    mesh = plsc.VectorSubcoreMesh(
        core_axis_name="c", subcore_axis_name="s",
        num_cores=NUM_CORES, num_subcores=NUM_SUBCORES)

    @functools.partial(
        pl.kernel,
        out_type=jax.ShapeDtypeStruct((hist, dt_n, bt_n, 8, 128),
                                      jnp.float32),
        mesh=mesh,
        scratch_types=[
            pltpu.VMEM((d * vocab,), jnp.float32),
            pltpu.VMEM((bt_per_w, hi_n, bi_n), jnp.int32),
            pltpu.VMEM((bt_per_w, hi_n, bi_n), jnp.int32),
            pltpu.VMEM((dt_n, 8, 128), jnp.float32),
            pltpu.VMEM((dt_n, 8, 128), jnp.float32),
            pltpu.SemaphoreType.DMA,
            pltpu.SemaphoreType.DMA,
            pltpu.SemaphoreType.DMA,
            pltpu.SemaphoreType.DMA,
        ],
        compiler_params=pltpu.CompilerParams(
            use_tc_tiling_on_sc=False, needs_layout_passes=False),
    )
    def k(tab_hbm, idx_hbm, out_hbm, tab_v, blk_a, blk_b, stage0, stage1,
          sem_a, sem_b, sem0, sem1):
        wid = lax.axis_index("s") * NUM_CORES + lax.axis_index("c")
        bt0 = wid * bt_per_w
        pltpu.sync_copy(tab_hbm, tab_v)

        def fire_idx(h8, blk, sem):
            pltpu.async_copy(
                idx_hbm.at[h8, pl.ds(bt0, bt_per_w)], blk, sem)

        def wait_idx(h8, blk, sem):
            pltpu.make_async_copy(
                idx_hbm.at[h8, pl.ds(bt0, bt_per_w)], blk, sem).wait()

        def fill(blk, btl, hi, stage):
            regs = [blk[btl, hi, pl.ds(g * LANES, LANES)] for g in range(8)]

            @plsc.parallel_loop(0, dt_n * 8, 1, unroll=4)
            def _(j):
                off = j * vocab
                vals = [plsc.load_gather(tab_v, [regs[g] + off])
                        for g in range(8)]
                for g in range(8):
                    stage[j // 8, j % 8, pl.ds(g * LANES, LANES)] = vals[g]

        def fire_st(h, btl, stage, sem):
            pltpu.async_copy(
                stage, out_hbm.at[h, pl.ds(0, dt_n), bt0 + btl], sem)

        def wait_st(h, btl, stage, sem):
            pltpu.make_async_copy(
                stage, out_hbm.at[h, pl.ds(0, dt_n), bt0 + btl], sem).wait()

        def block(h8, blk, first):
            def hi_body(hi, _):
                h = h8 * hi_n + hi
                skip = first & (hi == 0)

                @pl.when(jnp.logical_not(skip))
                def _():
                    wait_st(h, 0, stage0, sem0)
                fill(blk, 0, hi, stage0)
                fire_st(h, 0, stage0, sem0)

                @pl.when(jnp.logical_not(skip))
                def _():
                    wait_st(h, 1, stage1, sem1)
                fill(blk, 1, hi, stage1)
                fire_st(h, 1, stage1, sem1)

                wait_st(h, 2, stage0, sem0)
                fill(blk, 2, hi, stage0)
                fire_st(h, 2, stage0, sem0)

                wait_st(h, 3, stage1, sem1)
                fill(blk, 3, hi, stage1)
                fire_st(h, 3, stage1, sem1)
                return ()

            lax.fori_loop(0, hi_n, hi_body, (), unroll=False)

        fire_idx(0, blk_a, sem_a)

        def pair(p, _):
            ha = 2 * p
            hb = 2 * p + 1
            wait_idx(ha, blk_a, sem_a)
            fire_idx(hb, blk_b, sem_b)
            block(ha, blk_a, p == 0)
            wait_idx(hb, blk_b, sem_b)
            fire_idx(hb + 1, blk_a, sem_a)
            block(hb, blk_b, False)
            return ()

        lax.fori_loop(0, (h8_n - 1) // 2, pair, (), unroll=False)

        wait_idx(h8_n - 1, blk_a, sem_a)
        block(h8_n - 1, blk_a, False)
        wait_st(hist - 1, 2, stage0, sem0)
        wait_st(hist - 1, 3, stage1, sem1)

    return k(table_t, idx4)


def kernel(agent_types, table):
    b, h = agent_types.shape
    d = table.shape[1]
    idx4 = (agent_types.astype(jnp.int32)
            .reshape(b // 128, 128, h // 8, 8)
            .transpose(2, 0, 3, 1))
    table_t = table.T.reshape(d * table.shape[0])
    out5 = _embed(idx4, table_t)
    return out5.transpose(2, 4, 0, 1, 3).reshape(b, h, d)


# speedup vs baseline: 1.0041x; 1.0041x over previous
import functools

import jax
import jax.numpy as jnp
from jax import lax
from jax.experimental import pallas as pl
from jax.experimental.pallas import tpu as pltpu
from jax.experimental.pallas import tpu_sc as plsc

NUM_CORES = 2
NUM_SUBCORES = 16
NW = NUM_CORES * NUM_SUBCORES
LANES = 16


@jax.jit
def _embed(idx4, table_t):
    h8_n, bt_n, hi_n, bi_n = idx4.shape
    hist = h8_n * hi_n
    d = 64
    vocab = table_t.shape[0] // d
    dt_n = d // 8
    bt_per_w = bt_n // NW
    assert h8_n % 2 == 1 and h8_n >= 3 and bt_n % NW == 0

    mesh = plsc.VectorSubcoreMesh(
        core_axis_name="c", subcore_axis_name="s",
        num_cores=NUM_CORES, num_subcores=NUM_SUBCORES)

    @functools.partial(
        pl.kernel,
        out_type=jax.ShapeDtypeStruct((hist, dt_n, bt_n, 8, 128),
                                      jnp.float32),
        mesh=mesh,
        scratch_types=[
            pltpu.VMEM((d * vocab,), jnp.float32),
            pltpu.VMEM((bt_per_w, hi_n, bi_n), jnp.int32),
            pltpu.VMEM((bt_per_w, hi_n, bi_n), jnp.int32),
            pltpu.VMEM((dt_n, 8, 128), jnp.float32),
            pltpu.VMEM((dt_n, 8, 128), jnp.float32),
            pltpu.SemaphoreType.DMA,
            pltpu.SemaphoreType.DMA,
            pltpu.SemaphoreType.DMA,
            pltpu.SemaphoreType.DMA,
        ],
        compiler_params=pltpu.CompilerParams(
            use_tc_tiling_on_sc=False, needs_layout_passes=False),
    )
    def k(tab_hbm, idx_hbm, out_hbm, tab_v, blk_a, blk_b, stage0, stage1,
          sem_a, sem_b, sem0, sem1):
        wid = lax.axis_index("s") * NUM_CORES + lax.axis_index("c")
        bt0 = wid * bt_per_w
        pltpu.sync_copy(tab_hbm, tab_v)

        def fire_idx(h8, blk, sem):
            pltpu.async_copy(
                idx_hbm.at[h8, pl.ds(bt0, bt_per_w)], blk, sem)

        def wait_idx(h8, blk, sem):
            pltpu.make_async_copy(
                idx_hbm.at[h8, pl.ds(bt0, bt_per_w)], blk, sem).wait()

        def fill(blk, btl, hi, stage):
            regs = [blk[btl, hi, pl.ds(g * LANES, LANES)] for g in range(8)]

            @plsc.parallel_loop(0, dt_n * 8, 1, unroll=2)
            def _(j):
                off = j * vocab
                vals = [plsc.load_gather(tab_v, [regs[g] + off])
                        for g in range(8)]
                for g in range(8):
                    stage[j // 8, j % 8, pl.ds(g * LANES, LANES)] = vals[g]

        def fire_st(h, btl, stage, sem):
            pltpu.async_copy(
                stage, out_hbm.at[h, pl.ds(0, dt_n), bt0 + btl], sem)

        def wait_st(h, btl, stage, sem):
            pltpu.make_async_copy(
                stage, out_hbm.at[h, pl.ds(0, dt_n), bt0 + btl], sem).wait()

        def block(h8, blk, first):
            def hi_body(hi, _):
                h = h8 * hi_n + hi
                skip = first & (hi == 0)

                @pl.when(jnp.logical_not(skip))
                def _():
                    wait_st(h, 0, stage0, sem0)
                fill(blk, 0, hi, stage0)
                fire_st(h, 0, stage0, sem0)

                @pl.when(jnp.logical_not(skip))
                def _():
                    wait_st(h, 1, stage1, sem1)
                fill(blk, 1, hi, stage1)
                fire_st(h, 1, stage1, sem1)

                wait_st(h, 2, stage0, sem0)
                fill(blk, 2, hi, stage0)
                fire_st(h, 2, stage0, sem0)

                wait_st(h, 3, stage1, sem1)
                fill(blk, 3, hi, stage1)
                fire_st(h, 3, stage1, sem1)
                return ()

            lax.fori_loop(0, hi_n, hi_body, (), unroll=False)

        fire_idx(0, blk_a, sem_a)

        def pair(p, _):
            ha = 2 * p
            hb = 2 * p + 1
            wait_idx(ha, blk_a, sem_a)
            fire_idx(hb, blk_b, sem_b)
            block(ha, blk_a, p == 0)
            wait_idx(hb, blk_b, sem_b)
            fire_idx(hb + 1, blk_a, sem_a)
            block(hb, blk_b, False)
            return ()

        lax.fori_loop(0, (h8_n - 1) // 2, pair, (), unroll=False)

        wait_idx(h8_n - 1, blk_a, sem_a)
        block(h8_n - 1, blk_a, False)
        wait_st(hist - 1, 2, stage0, sem0)
        wait_st(hist - 1, 3, stage1, sem1)

    return k(table_t, idx4)


def kernel(agent_types, table):
    b, h = agent_types.shape
    d = table.shape[1]
    idx4 = (agent_types.astype(jnp.int32)
            .reshape(b // 128, 128, h // 8, 8)
            .transpose(2, 0, 3, 1))
    table_t = table.T.reshape(d * table.shape[0])
    out5 = _embed(idx4, table_t)
    return out5.transpose(2, 4, 0, 1, 3).reshape(b, h, d)
